# baseline (speedup 1.0000x reference)
import functools

import jax
import jax.numpy as jnp
from jax import lax
from jax.experimental import pallas as pl
from jax.experimental.pallas import tpu as pltpu
from jax.experimental.pallas import tpu_sc as plsc

H = 128
BLK = 1000
F32 = jnp.float32
I32 = jnp.int32


def _segsum_multi(cfgs):
    mesh = plsc.VectorSubcoreMesh(core_axis_name="c", subcore_axis_name="s")
    n_et = len(cfgs)
    max_ibc = max(cfg[3] for cfg in cfgs)
    max_ndp = max(cfg[0] for cfg in cfgs)

    @functools.partial(
        pl.kernel,
        mesh=mesh,
        out_type=tuple(jax.ShapeDtypeStruct((4, cfg[0], 32), F32)
                       for cfg in cfgs),
        scratch_types=[
            pltpu.VMEM((max_ibc, 128), I32),
            pltpu.VMEM((max_ibc, 128), I32),
            pltpu.VMEM((128, 32), F32),
            pltpu.VMEM((128, 32), F32),
            pltpu.VMEM_SHARED((max_ndp, 32), F32),
            pltpu.SemaphoreType.DMA,
            pltpu.SemaphoreType.DMA,
        ],
        compiler_params=pltpu.CompilerParams(use_tc_tiling_on_sc=False),
    )
    def k(*refs):
        ins = refs[:3 * n_et]
        outs = refs[3 * n_et:4 * n_et]
        (src_v, dst_v, r0, r1, acc, g0, g1) = refs[4 * n_et:]
        c = lax.axis_index("c")
        s = lax.axis_index("s")
        rows = [r0, r1]
        gsem = [g0, g1]
        zeros16 = jnp.zeros((16,), F32)

        for ei in range(n_et):
            nd_pad, nch, q, ibc = cfgs[ei]
            xs_hbm, src4_hbm, dst3_hbm = ins[3 * ei:3 * ei + 3]
            out_hbm = outs[ei]
            nblocks = nch // ibc
            row0 = s * q
            nz_full, nz_tail = q // 128, q % 128

            def fire_g(u, j):
                pltpu.async_copy(xs_hbm.at[src_v.at[j]], rows[u], gsem[u])

            def drain_g(u):
                pltpu.make_async_copy(xs_hbm.at[src_v.at[0]], rows[u],
                                      gsem[u]).wait()

            def scat(u, j):
                pltpu.sync_copy(rows[u], acc.at[dst_v.at[j]], add=True)

            for p in range(2):
                strip = 2 * c + p

                def zfill(i, carry):
                    r0[i, pl.ds(0, 16)] = zeros16
                    r0[i, pl.ds(16, 16)] = zeros16
                    return carry

                lax.fori_loop(0, 128, zfill, 0)

                def zcopy(i, carry):
                    pltpu.sync_copy(r0, acc.at[pl.ds(row0 + i * 128, 128)])
                    return carry

                lax.fori_loop(0, nz_full, zcopy, 0)
                if nz_tail:
                    pltpu.sync_copy(
                        r0.at[pl.ds(0, nz_tail)],
                        acc.at[pl.ds(row0 + nz_full * 128, nz_tail)])
                plsc.subcore_barrier()

                def block(b, carry):
                    pltpu.sync_copy(
                        src4_hbm.at[strip * 16 + s, pl.ds(b * ibc, ibc)],
                        src_v.at[pl.ds(0, ibc)])
                    pltpu.sync_copy(dst3_hbm.at[s, pl.ds(b * ibc, ibc)],
                                    dst_v.at[pl.ds(0, ibc)])
                    fire_g(0, 0)

                    def pair(t, carry2):
                        j1 = 2 * t + 1
                        j2 = jnp.minimum(2 * t + 2, ibc - 1)
                        fire_g(1, j1)
                        drain_g(0)
                        scat(0, 2 * t)
                        fire_g(0, j2)
                        drain_g(1)
                        scat(1, j1)
                        return carry2

                    lax.fori_loop(0, ibc // 2, pair, 0)
                    drain_g(0)
                    return carry

                lax.fori_loop(0, nblocks, block, 0)
                plsc.subcore_barrier()
                pltpu.sync_copy(acc.at[pl.ds(row0, q)],
                                out_hbm.at[strip, pl.ds(row0, q)])
                if not (ei == n_et - 1 and p == 1):
                    plsc.subcore_barrier()

    return k


def _prep_edges(ei, s_rows, nd, nch):
    e = ei.shape[1]
    ep = 16 * nch * 128
    src = jnp.concatenate([ei[0], jnp.zeros((ep - e,), I32)])
    dst = jnp.concatenate([ei[1], jnp.full((ep - e,), nd, I32)])
    dst3 = dst.reshape(16, nch, 128)
    src_r = src.reshape(1, 16, nch, 128)
    offs = (jnp.arange(4, dtype=I32) * s_rows).reshape(4, 1, 1, 1)
    src4 = (src_r + offs).reshape(64, nch, 128)
    return src4, dst3


def _seg_dims(nd, e):
    per = -(-e // 16)
    nch0 = -(-per // 128)
    nch = ibc = None
    for cand_nch in range(nch0, nch0 + 33):
        best = 0
        for div in range(4, 53, 4):
            if cand_nch % div == 0:
                best = div
        if best:
            nch, ibc = cand_nch, best
            break
    q = -(-(nd + 1) // (16 * 64)) * 64
    return 16 * q, nch, q, ibc


def _full(shape):
    return pl.BlockSpec(shape, lambda i: tuple(0 for _ in shape))


def _strip_spec(cap=None):
    if cap is None:
        return pl.BlockSpec((4, BLK, 32), lambda i: (0, i, 0))
    return pl.BlockSpec((4, BLK, 32), lambda i: (0, jnp.minimum(i, cap - 1), 0))


def _cat4(ref):
    return jnp.concatenate([ref[p] for p in range(4)], axis=-1)


def _mlp(x, w1, b1, w2, b2):
    a = jnp.maximum(jnp.dot(x, w1, preferred_element_type=F32) + b1, 0.0)
    return jnp.dot(a, w2, preferred_element_type=F32) + b2


def _proj_call(x, w, b, n):
    def body(x_ref, w_ref, b_ref, o_ref):
        y = jnp.maximum(
            jnp.dot(x_ref[...], w_ref[...], preferred_element_type=F32)
            + b_ref[...], 0.0)
        for p in range(4):
            o_ref[p] = y[:, 32 * p:32 * p + 32]

    return pl.pallas_call(
        body,
        grid=(n // BLK,),
        in_specs=[pl.BlockSpec((BLK, H), lambda i: (i, 0)),
                  _full((H, H)), _full((1, H))],
        out_specs=_strip_spec(),
        out_shape=jax.ShapeDtypeStruct((4, n, 32), F32),
    )(x, w, b.reshape(1, H))


def _gin_call(x4, branches, n):
    nb = len(branches)
    caps = [br[1] for br in branches]

    def body(*refs):
        x4_ref = refs[0]
        m_refs = refs[1:1 + nb]
        w_refs = refs[1 + nb:1 + nb + 4 * nb]
        o_ref = refs[-1]
        i = pl.program_id(0)
        x = _cat4(x4_ref)
        tot = None
        for bi in range(nb):
            m = _cat4(m_refs[bi])
            if caps[bi] is not None:
                m = m * (i < caps[bi]).astype(F32)
            w1, b1, w2, b2 = w_refs[4 * bi:4 * bi + 4]
            y = _mlp(x + m, w1[...], b1[...], w2[...], b2[...])
            tot = y if tot is None else tot + y
        y = jnp.maximum(tot, 0.0)
        for p in range(4):
            o_ref[p] = y[:, 32 * p:32 * p + 32]

    in_specs = [_strip_spec()]
    args = [x4]
    for (m4, cap, w1, b1, w2, b2) in branches:
        in_specs.append(_strip_spec(cap))
        args.append(m4)
    for (m4, cap, w1, b1, w2, b2) in branches:
        in_specs += [_full((H, H)), _full((1, H)), _full((H, H)),
                     _full((1, H))]
        args += [w1, b1.reshape(1, H), w2, b2.reshape(1, H)]

    return pl.pallas_call(
        body,
        grid=(n // BLK,),
        in_specs=in_specs,
        out_specs=_strip_spec(),
        out_shape=jax.ShapeDtypeStruct((4, n, 32), F32),
    )(*args)


def _gin_out_call(x4, branches, wo, bo, n):
    nb = len(branches)
    caps = [br[1] for br in branches]

    def body(*refs):
        x4_ref = refs[0]
        m_refs = refs[1:1 + nb]
        w_refs = refs[1 + nb:1 + nb + 4 * nb]
        wo_ref, bo_ref = refs[1 + 5 * nb:3 + 5 * nb]
        o_ref = refs[-1]
        i = pl.program_id(0)
        x = _cat4(x4_ref)
        tot = None
        for bi in range(nb):
            m = _cat4(m_refs[bi])
            if caps[bi] is not None:
                m = m * (i < caps[bi]).astype(F32)
            w1, b1, w2, b2 = w_refs[4 * bi:4 * bi + 4]
            y = _mlp(x + m, w1[...], b1[...], w2[...], b2[...])
            tot = y if tot is None else tot + y
        y = jnp.maximum(tot, 0.0)
        o_ref[...] = (jnp.sum(y * wo_ref[...], axis=1)
                      + bo_ref[0, 0])[:, None]

    in_specs = [_strip_spec()]
    args = [x4]
    for (m4, cap, w1, b1, w2, b2) in branches:
        in_specs.append(_strip_spec(cap))
        args.append(m4)
    for (m4, cap, w1, b1, w2, b2) in branches:
        in_specs += [_full((H, H)), _full((1, H)), _full((H, H)),
                     _full((1, H))]
        args += [w1, b1.reshape(1, H), w2, b2.reshape(1, H)]
    in_specs += [_full((1, H)), _full((1, 1))]
    args += [wo.reshape(1, H), bo.reshape(1, 1)]

    return pl.pallas_call(
        body,
        grid=(n // BLK,),
        in_specs=in_specs,
        out_specs=pl.BlockSpec((BLK, 1), lambda i: (i, 0)),
        out_shape=jax.ShapeDtypeStruct((n, 1), F32),
    )(*args).reshape(n)


N_USER, N_PROD, N_CAT = 50000, 100000, 5000


def kernel(x_user, x_product, x_category, ei_up, ei_pu, ei_pc, ei_cp, params):
    p = params

    d_up = _seg_dims(50000, 200000)
    d_pu = _seg_dims(50000, 200000)
    d_pc = _seg_dims(5000, 100000)
    d_cp = _seg_dims(5000, 100000)

    e_up = _prep_edges(ei_up, N_USER, 50000, d_up[1])
    e_pu = _prep_edges(ei_pu, N_PROD, 50000, d_pu[1])
    e_pc = _prep_edges(ei_pc, N_PROD, 5000, d_pc[1])
    e_cp = _prep_edges(ei_cp, N_CAT, 5000, d_cp[1])

    seg_l0 = _segsum_multi([d_up, d_cp, d_pu, d_pc])
    seg_l1 = _segsum_multi([d_up, d_cp])

    h_user = _proj_call(x_user, p['proj_user_W'], p['proj_user_b'], N_USER)
    h_prod = _proj_call(x_product, p['proj_product_W'], p['proj_product_b'],
                        N_PROD)
    h_cat = _proj_call(x_category, p['proj_category_W'], p['proj_category_b'],
                       N_CAT)

    def W(l, et):
        return (p['l%d_%s_W1' % (l, et)], p['l%d_%s_b1' % (l, et)],
                p['l%d_%s_W2' % (l, et)], p['l%d_%s_b2' % (l, et)])

    def flat(h4):
        return h4.reshape(-1, 32)

    m_up, m_cp, m_pu, m_pc = seg_l0(
        flat(h_user), *e_up, flat(h_cat), *e_cp,
        flat(h_prod), *e_pu, flat(h_prod), *e_pc)

    h_prod1 = _gin_call(
        h_prod,
        [(m_up, 50, *W(0, 'up')), (m_cp, 5, *W(0, 'cp'))],
        N_PROD)
    h_user1 = _gin_call(h_user, [(m_pu, None, *W(0, 'pu'))], N_USER)
    h_cat1 = _gin_call(h_cat, [(m_pc, None, *W(0, 'pc'))], N_CAT)

    m_up1, m_cp1 = seg_l1(flat(h_user1), *e_up, flat(h_cat1), *e_cp)

    out = _gin_out_call(
        h_prod1,
        [(m_up1, 50, *W(1, 'up')), (m_cp1, 5, *W(1, 'cp'))],
        p['out_W'], p['out_b'], N_PROD)
    return out


# speedup vs baseline: 1.1540x; 1.1540x over previous
import functools

import jax
import jax.numpy as jnp
from jax import lax
from jax.experimental import pallas as pl
from jax.experimental.pallas import tpu as pltpu
from jax.experimental.pallas import tpu_sc as plsc

H = 128
BLK = 1000
F32 = jnp.float32
I32 = jnp.int32


def _segsum_multi(cfgs):
    mesh = plsc.VectorSubcoreMesh(core_axis_name="c", subcore_axis_name="s")
    n_et = len(cfgs)
    max_ibc = max(cfg[3] for cfg in cfgs)
    max_ndp = max(cfg[0] for cfg in cfgs)

    @functools.partial(
        pl.kernel,
        mesh=mesh,
        out_type=tuple(jax.ShapeDtypeStruct((4, cfg[0], 32), F32)
                       for cfg in cfgs),
        scratch_types=[
            pltpu.VMEM((max_ibc, 128), I32),
            pltpu.VMEM((max_ibc, 128), I32),
            pltpu.VMEM((128, 32), F32),
            pltpu.VMEM((128, 32), F32),
            pltpu.VMEM_SHARED((max_ndp, 32), F32),
            pltpu.SemaphoreType.DMA,
            pltpu.SemaphoreType.DMA,
        ],
        compiler_params=pltpu.CompilerParams(use_tc_tiling_on_sc=False),
    )
    def k(*refs):
        ins = refs[:3 * n_et]
        outs = refs[3 * n_et:4 * n_et]
        (src_v, dst_v, r0, r1, acc, g0, g1) = refs[4 * n_et:]
        c = lax.axis_index("c")
        s = lax.axis_index("s")
        rows = [r0, r1]
        gsem = [g0, g1]
        zeros16 = jnp.zeros((16,), F32)

        for ei in range(n_et):
            nd_pad, nch, q, ibc = cfgs[ei]
            xs_hbm, src4_hbm, dst3_hbm = ins[3 * ei:3 * ei + 3]
            out_hbm = outs[ei]
            nblocks = nch // ibc
            row0 = s * q
            nz_full, nz_tail = q // 128, q % 128

            def fire_g(u, j):
                pltpu.async_copy(xs_hbm.at[src_v.at[j]], rows[u], gsem[u])

            def drain_g(u):
                pltpu.make_async_copy(xs_hbm.at[src_v.at[0]], rows[u],
                                      gsem[u]).wait()

            def scat(u, j):
                pltpu.sync_copy(rows[u], acc.at[dst_v.at[j]], add=True)

            for p in range(2):
                strip = 2 * c + p

                def zfill(i, carry):
                    r0[i, pl.ds(0, 16)] = zeros16
                    r0[i, pl.ds(16, 16)] = zeros16
                    return carry

                lax.fori_loop(0, 128, zfill, 0)

                def zcopy(i, carry):
                    pltpu.sync_copy(r0, acc.at[pl.ds(row0 + i * 128, 128)])
                    return carry

                lax.fori_loop(0, nz_full, zcopy, 0)
                if nz_tail:
                    pltpu.sync_copy(
                        r0.at[pl.ds(0, nz_tail)],
                        acc.at[pl.ds(row0 + nz_full * 128, nz_tail)])
                plsc.subcore_barrier()

                def block(b, carry):
                    pltpu.sync_copy(
                        src4_hbm.at[strip * 16 + s, pl.ds(b * ibc, ibc)],
                        src_v.at[pl.ds(0, ibc)])
                    pltpu.sync_copy(dst3_hbm.at[s, pl.ds(b * ibc, ibc)],
                                    dst_v.at[pl.ds(0, ibc)])
                    fire_g(0, 0)

                    def pair(t, carry2):
                        j1 = 2 * t + 1
                        j2 = jnp.minimum(2 * t + 2, ibc - 1)
                        fire_g(1, j1)
                        drain_g(0)
                        scat(0, 2 * t)
                        fire_g(0, j2)
                        drain_g(1)
                        scat(1, j1)
                        return carry2

                    lax.fori_loop(0, ibc // 2, pair, 0)
                    drain_g(0)
                    return carry

                lax.fori_loop(0, nblocks, block, 0)
                plsc.subcore_barrier()
                pltpu.sync_copy(acc.at[pl.ds(row0, q)],
                                out_hbm.at[strip, pl.ds(row0, q)])
                if not (ei == n_et - 1 and p == 1):
                    plsc.subcore_barrier()

    return k


def _prep_edges(ei, s_rows, nd, nch):
    e = ei.shape[1]
    ep = 16 * nch * 128
    src = jnp.concatenate([ei[0], jnp.zeros((ep - e,), I32)])
    dst = jnp.concatenate([ei[1], jnp.full((ep - e,), nd, I32)])
    dst3 = dst.reshape(16, nch, 128)
    src_r = src.reshape(1, 16, nch, 128)
    offs = (jnp.arange(4, dtype=I32) * s_rows).reshape(4, 1, 1, 1)
    src4 = (src_r + offs).reshape(64, nch, 128)
    return src4, dst3


def _seg_dims(nd, e):
    per = -(-e // 16)
    nch0 = -(-per // 128)
    nch = ibc = None
    for cand_nch in range(nch0, nch0 + 33):
        best = 0
        for div in range(4, 53, 4):
            if cand_nch % div == 0:
                best = div
        if best:
            nch, ibc = cand_nch, best
            break
    q = -(-(nd + 1) // (16 * 64)) * 64
    return 16 * q, nch, q, ibc


def _full(shape):
    return pl.BlockSpec(shape, lambda i: tuple(0 for _ in shape))


def _strip_spec(off=0):
    return pl.BlockSpec((4, BLK, 32), lambda i: (0, i + off, 0))


def _cat4(ref):
    return jnp.concatenate([ref[p] for p in range(4)], axis=-1)


BF16 = jnp.bfloat16


def _dot3(x, w):
    return jnp.dot(x.astype(BF16), w.astype(BF16),
                   preferred_element_type=F32)


def _mlp(x, w1, b1, w2, b2):
    a = jnp.maximum(_dot3(x, w1) + b1, 0.0)
    return _dot3(a, w2) + b2


def _proj_call(x, w, b, n):
    def body(x_ref, w_ref, b_ref, o_ref):
        y = jnp.maximum(_dot3(x_ref[...], w_ref[...]) + b_ref[...], 0.0)
        for p in range(4):
            o_ref[p] = y[:, 32 * p:32 * p + 32]

    return pl.pallas_call(
        body,
        grid=(n // BLK,),
        in_specs=[pl.BlockSpec((BLK, H), lambda i: (i, 0)),
                  _full((H, H)), _full((1, H))],
        out_specs=_strip_spec(),
        out_shape=jax.ShapeDtypeStruct((4, n, 32), F32),
    )(x, w, b.reshape(1, H))


def _gin_common(x4, branches, n, x_off, m_off, head):
    nb = len(branches)
    has_m = [br[0] is not None for br in branches]
    nm = sum(has_m)

    def body(*refs):
        x4_ref = refs[0]
        m_refs = refs[1:1 + nm]
        w_refs = refs[1 + nm:1 + nm + 4 * nb]
        o_ref = refs[-1]
        x = _cat4(x4_ref)
        tot = None
        mi = 0
        for bi in range(nb):
            h = x
            if has_m[bi]:
                h = x + _cat4(m_refs[mi])
                mi += 1
            w1, b1, w2, b2 = w_refs[4 * bi:4 * bi + 4]
            y = _mlp(h, w1[...], b1[...], w2[...], b2[...])
            tot = y if tot is None else tot + y
        y = jnp.maximum(tot, 0.0)
        if head is None:
            for p in range(4):
                o_ref[p] = y[:, 32 * p:32 * p + 32]
        else:
            wo_ref, bo_ref = refs[1 + nm + 4 * nb:3 + nm + 4 * nb]
            yb = y.astype(BF16).astype(F32)
            wb = wo_ref[...].astype(BF16).astype(F32)
            o_ref[...] = (jnp.sum(yb * wb, axis=1)
                          + bo_ref[0, 0])[:, None]

    in_specs = [_strip_spec(x_off)]
    args = [x4]
    for br in branches:
        if br[0] is not None:
            in_specs.append(_strip_spec(m_off))
            args.append(br[0])
    for (m4, w1, b1, w2, b2) in branches:
        in_specs += [_full((H, H)), _full((1, H)), _full((H, H)),
                     _full((1, H))]
        args += [w1, b1.reshape(1, H), w2, b2.reshape(1, H)]
    if head is None:
        out_specs = _strip_spec()
        out_shape = jax.ShapeDtypeStruct((4, n, 32), F32)
    else:
        wo, bo = head
        in_specs += [_full((1, H)), _full((1, 1))]
        args += [wo.reshape(1, H), bo.reshape(1, 1)]
        out_specs = pl.BlockSpec((BLK, 1), lambda i: (i, 0))
        out_shape = jax.ShapeDtypeStruct((n, 1), F32)

    out = pl.pallas_call(
        body,
        grid=(n // BLK,),
        in_specs=in_specs,
        out_specs=out_specs,
        out_shape=out_shape,
    )(*args)
    return out if head is None else out.reshape(n)


def _gin_call(x4, branches, n, x_off=0, m_off=0):
    return _gin_common(x4, branches, n, x_off, m_off, None)


def _gin_out_call(x4, branches, wo, bo, n, x_off=0, m_off=0):
    return _gin_common(x4, branches, n, x_off, m_off, (wo, bo))


N_USER, N_PROD, N_CAT = 50000, 100000, 5000


def kernel(x_user, x_product, x_category, ei_up, ei_pu, ei_pc, ei_cp, params):
    p = params

    d_up = _seg_dims(50000, 200000)
    d_pu = _seg_dims(50000, 200000)
    d_pc = _seg_dims(5000, 100000)
    d_cp = _seg_dims(5000, 100000)

    e_up = _prep_edges(ei_up, N_USER, 50000, d_up[1])
    e_pu = _prep_edges(ei_pu, N_PROD, 50000, d_pu[1])
    e_pc = _prep_edges(ei_pc, N_PROD, 5000, d_pc[1])
    e_cp = _prep_edges(ei_cp, N_CAT, 5000, d_cp[1])

    seg_up = _segsum_multi([d_up])
    seg_cp = _segsum_multi([d_cp])
    seg_pu = _segsum_multi([d_pu])
    seg_pc = _segsum_multi([d_pc])

    h_user = _proj_call(x_user, p['proj_user_W'], p['proj_user_b'], N_USER)
    h_prod = _proj_call(x_product, p['proj_product_W'], p['proj_product_b'],
                        N_PROD)
    h_cat = _proj_call(x_category, p['proj_category_W'], p['proj_category_b'],
                       N_CAT)

    def W(l, et):
        return (p['l%d_%s_W1' % (l, et)], p['l%d_%s_b1' % (l, et)],
                p['l%d_%s_W2' % (l, et)], p['l%d_%s_b2' % (l, et)])

    def flat(h4):
        return h4.reshape(-1, 32)

    (m_up,) = seg_up(flat(h_user), *e_up)
    (m_cp,) = seg_cp(flat(h_cat), *e_cp)
    (m_pu,) = seg_pu(flat(h_prod), *e_pu)
    (m_pc,) = seg_pc(flat(h_prod), *e_pc)

    wu0, wc0 = W(0, 'up'), W(0, 'cp')
    h1p_a = _gin_call(h_prod, [(m_up, *wu0), (m_cp, *wc0)], 5000)
    h1p_b = _gin_call(h_prod, [(m_up, *wu0), (None, *wc0)], 45000,
                      x_off=5, m_off=5)
    h1p_c = _gin_call(h_prod, [(None, *wu0), (None, *wc0)], 50000, x_off=50)
    h_user1 = _gin_call(h_user, [(m_pu, *W(0, 'pu'))], N_USER)
    h_cat1 = _gin_call(h_cat, [(m_pc, *W(0, 'pc'))], N_CAT)

    (m_up1,) = seg_up(flat(h_user1), *e_up)
    (m_cp1,) = seg_cp(flat(h_cat1), *e_cp)

    wu1, wc1 = W(1, 'up'), W(1, 'cp')
    wo, bo = p['out_W'], p['out_b']
    out_a = _gin_out_call(h1p_a, [(m_up1, *wu1), (m_cp1, *wc1)], wo, bo, 5000)
    out_b = _gin_out_call(h1p_b, [(m_up1, *wu1), (None, *wc1)], wo, bo,
                          45000, m_off=5)
    out_c = _gin_out_call(h1p_c, [(None, *wu1), (None, *wc1)], wo, bo, 50000)
    return jnp.concatenate([out_a, out_b, out_c])


# speedup vs baseline: 1.5342x; 1.3294x over previous
import functools

import jax
import jax.numpy as jnp
from jax import lax
from jax.experimental import pallas as pl
from jax.experimental.pallas import tpu as pltpu
from jax.experimental.pallas import tpu_sc as plsc

H = 128
BLK = 1000
F32 = jnp.float32
I32 = jnp.int32


def _segsum_multi(cfgs):
    mesh = plsc.VectorSubcoreMesh(core_axis_name="c", subcore_axis_name="s")
    n_et = len(cfgs)
    max_ibc = max(cfg[3] for cfg in cfgs)
    max_ndp = max(cfg[0] for cfg in cfgs)

    @functools.partial(
        pl.kernel,
        mesh=mesh,
        out_type=tuple(jax.ShapeDtypeStruct((4, cfg[0], 32), F32)
                       for cfg in cfgs),
        scratch_types=[
            pltpu.VMEM((max_ibc, 128), I32),
            pltpu.VMEM((max_ibc, 128), I32),
            pltpu.VMEM((128, 32), F32),
            pltpu.VMEM((128, 32), F32),
            pltpu.VMEM((128, 32), F32),
            pltpu.VMEM_SHARED((max_ndp, 32), F32),
            pltpu.SemaphoreType.DMA,
            pltpu.SemaphoreType.DMA,
        ],
        compiler_params=pltpu.CompilerParams(use_tc_tiling_on_sc=False),
    )
    def k(*refs):
        ins = refs[:3 * n_et]
        outs = refs[3 * n_et:4 * n_et]
        (src_v, dst_v, r0, r1, zer_v, acc, g0, g1) = refs[4 * n_et:]
        c = lax.axis_index("c")
        s = lax.axis_index("s")
        rows = [r0, r1]
        gsem = [g0, g1]
        zeros16 = jnp.zeros((16,), F32)

        def zfill(i, carry):
            zer_v[i, pl.ds(0, 16)] = zeros16
            zer_v[i, pl.ds(16, 16)] = zeros16
            return carry

        lax.fori_loop(0, 128, zfill, 0)

        for ei in range(n_et):
            nd_pad, nch, q, ibc = cfgs[ei]
            xs_hbm, src4_hbm, dst3_hbm = ins[3 * ei:3 * ei + 3]
            out_hbm = outs[ei]
            nblocks = nch // ibc
            row0 = s * q
            nz_full, nz_tail = q // 128, q % 128

            def fire_g(u, j):
                pltpu.async_copy(xs_hbm.at[src_v.at[j]], rows[u], gsem[u])

            def drain_g(u):
                pltpu.make_async_copy(xs_hbm.at[src_v.at[0]], rows[u],
                                      gsem[u]).wait()

            def scat(u, j):
                pltpu.sync_copy(rows[u], acc.at[dst_v.at[j]], add=True)

            for p in range(2):
                strip = 2 * c + p

                def zcopy(i, carry):
                    pltpu.sync_copy(zer_v, acc.at[pl.ds(row0 + i * 128, 128)])
                    return carry

                lax.fori_loop(0, nz_full, zcopy, 0)
                if nz_tail:
                    pltpu.sync_copy(
                        zer_v.at[pl.ds(0, nz_tail)],
                        acc.at[pl.ds(row0 + nz_full * 128, nz_tail)])
                plsc.subcore_barrier()

                def block(b, carry):
                    pltpu.sync_copy(
                        src4_hbm.at[strip * 16 + s, pl.ds(b * ibc, ibc)],
                        src_v.at[pl.ds(0, ibc)])
                    pltpu.sync_copy(dst3_hbm.at[s, pl.ds(b * ibc, ibc)],
                                    dst_v.at[pl.ds(0, ibc)])
                    fire_g(0, 0)

                    def pair(t, carry2):
                        j1 = 2 * t + 1
                        j2 = jnp.minimum(2 * t + 2, ibc - 1)
                        fire_g(1, j1)
                        drain_g(0)
                        scat(0, 2 * t)
                        fire_g(0, j2)
                        drain_g(1)
                        scat(1, j1)
                        return carry2

                    lax.fori_loop(0, ibc // 2, pair, 0)
                    drain_g(0)
                    return carry

                lax.fori_loop(0, nblocks, block, 0)
                plsc.subcore_barrier()
                pltpu.sync_copy(acc.at[pl.ds(row0, q)],
                                out_hbm.at[strip, pl.ds(row0, q)])
                if not (ei == n_et - 1 and p == 1):
                    plsc.subcore_barrier()

    return k


def _prep_edges(ei, s_rows, nd, nch):
    e = ei.shape[1]
    ep = 16 * nch * 128
    src = jnp.concatenate([ei[0], jnp.zeros((ep - e,), I32)])
    dst = jnp.concatenate([ei[1], jnp.full((ep - e,), nd, I32)])
    dst3 = dst.reshape(16, nch, 128)
    src_r = src.reshape(1, 16, nch, 128)
    offs = (jnp.arange(4, dtype=I32) * s_rows).reshape(4, 1, 1, 1)
    src4 = (src_r + offs).reshape(64, nch, 128)
    return src4, dst3


def _seg_dims(nd, e):
    per = -(-e // 16)
    nch0 = -(-per // 128)
    nch = ibc = None
    for cand_nch in range(nch0, nch0 + 17):
        for cand in range(16, 1, -2):
            if cand_nch % cand == 0:
                nch, ibc = cand_nch, cand
                break
        if nch is not None:
            break
    q = -(-(nd + 1) // (16 * 128)) * 128
    return 16 * q, nch, q, ibc


def _full(shape):
    return pl.BlockSpec(shape, lambda i: tuple(0 for _ in shape))


def _strip_spec(off=0):
    return pl.BlockSpec((4, BLK, 32), lambda i: (0, i + off, 0))


def _cat4(ref):
    return jnp.concatenate([ref[p] for p in range(4)], axis=-1)


BF16 = jnp.bfloat16


def _dot3(x, w):
    return jnp.dot(x.astype(BF16), w.astype(BF16),
                   preferred_element_type=F32)


def _mlp(x, w1, b1, w2, b2):
    a = jnp.maximum(_dot3(x, w1) + b1, 0.0)
    return _dot3(a, w2) + b2


def _proj_call(x, w, b, n):
    def body(x_ref, w_ref, b_ref, o_ref):
        y = jnp.maximum(_dot3(x_ref[...], w_ref[...]) + b_ref[...], 0.0)
        for p in range(4):
            o_ref[p] = y[:, 32 * p:32 * p + 32]

    return pl.pallas_call(
        body,
        grid=(n // BLK,),
        in_specs=[pl.BlockSpec((BLK, H), lambda i: (i, 0)),
                  _full((H, H)), _full((1, H))],
        out_specs=_strip_spec(),
        out_shape=jax.ShapeDtypeStruct((4, n, 32), F32),
    )(x, w, b.reshape(1, H))


def _gin_common(x4, branches, n, x_off, m_off, head):
    nb = len(branches)
    has_m = [br[0] is not None for br in branches]
    nm = sum(has_m)

    def body(*refs):
        x4_ref = refs[0]
        m_refs = refs[1:1 + nm]
        w_refs = refs[1 + nm:1 + nm + 4 * nb]
        o_ref = refs[-1]
        x = _cat4(x4_ref)
        tot = None
        mi = 0
        for bi in range(nb):
            h = x
            if has_m[bi]:
                h = x + _cat4(m_refs[mi])
                mi += 1
            w1, b1, w2, b2 = w_refs[4 * bi:4 * bi + 4]
            y = _mlp(h, w1[...], b1[...], w2[...], b2[...])
            tot = y if tot is None else tot + y
        y = jnp.maximum(tot, 0.0)
        if head is None:
            for p in range(4):
                o_ref[p] = y[:, 32 * p:32 * p + 32]
        else:
            wo_ref, bo_ref = refs[1 + nm + 4 * nb:3 + nm + 4 * nb]
            yb = y.astype(BF16).astype(F32)
            wb = wo_ref[...].astype(BF16).astype(F32)
            o_ref[...] = (jnp.sum(yb * wb, axis=1)
                          + bo_ref[0, 0])[:, None]

    in_specs = [_strip_spec(x_off)]
    args = [x4]
    for br in branches:
        if br[0] is not None:
            in_specs.append(_strip_spec(m_off))
            args.append(br[0])
    for (m4, w1, b1, w2, b2) in branches:
        in_specs += [_full((H, H)), _full((1, H)), _full((H, H)),
                     _full((1, H))]
        args += [w1, b1.reshape(1, H), w2, b2.reshape(1, H)]
    if head is None:
        out_specs = _strip_spec()
        out_shape = jax.ShapeDtypeStruct((4, n, 32), F32)
    else:
        wo, bo = head
        in_specs += [_full((1, H)), _full((1, 1))]
        args += [wo.reshape(1, H), bo.reshape(1, 1)]
        out_specs = pl.BlockSpec((BLK, 1), lambda i: (i, 0))
        out_shape = jax.ShapeDtypeStruct((n, 1), F32)

    out = pl.pallas_call(
        body,
        grid=(n // BLK,),
        in_specs=in_specs,
        out_specs=out_specs,
        out_shape=out_shape,
    )(*args)
    return out if head is None else out.reshape(n)


def _gin_call(x4, branches, n, x_off=0, m_off=0):
    return _gin_common(x4, branches, n, x_off, m_off, None)


def _gin_out_call(x4, branches, wo, bo, n, x_off=0, m_off=0):
    return _gin_common(x4, branches, n, x_off, m_off, (wo, bo))


N_USER, N_PROD, N_CAT = 50000, 100000, 5000


def kernel(x_user, x_product, x_category, ei_up, ei_pu, ei_pc, ei_cp, params):
    p = params

    d_up = _seg_dims(50000, 200000)
    d_pu = _seg_dims(50000, 200000)
    d_pc = _seg_dims(5000, 100000)
    d_cp = _seg_dims(5000, 100000)

    e_up = _prep_edges(ei_up, N_USER, 50000, d_up[1])
    e_pu = _prep_edges(ei_pu, N_PROD, 50000, d_pu[1])
    e_pc = _prep_edges(ei_pc, N_PROD, 5000, d_pc[1])
    e_cp = _prep_edges(ei_cp, N_CAT, 5000, d_cp[1])

    seg_up = _segsum_multi([d_up])
    seg_cp = _segsum_multi([d_cp])
    seg_pu = _segsum_multi([d_pu])
    seg_pc = _segsum_multi([d_pc])

    h_user = _proj_call(x_user, p['proj_user_W'], p['proj_user_b'], N_USER)
    h_prod = _proj_call(x_product, p['proj_product_W'], p['proj_product_b'],
                        N_PROD)
    h_cat = _proj_call(x_category, p['proj_category_W'], p['proj_category_b'],
                       N_CAT)

    def W(l, et):
        return (p['l%d_%s_W1' % (l, et)], p['l%d_%s_b1' % (l, et)],
                p['l%d_%s_W2' % (l, et)], p['l%d_%s_b2' % (l, et)])

    def flat(h4):
        return h4.reshape(-1, 32)

    (m_up,) = seg_up(flat(h_user), *e_up)
    (m_cp,) = seg_cp(flat(h_cat), *e_cp)
    (m_pu,) = seg_pu(flat(h_prod), *e_pu)
    (m_pc,) = seg_pc(flat(h_prod), *e_pc)

    wu0, wc0 = W(0, 'up'), W(0, 'cp')
    h1p_a = _gin_call(h_prod, [(m_up, *wu0), (m_cp, *wc0)], 5000)
    h1p_b = _gin_call(h_prod, [(m_up, *wu0), (None, *wc0)], 45000,
                      x_off=5, m_off=5)
    h1p_c = _gin_call(h_prod, [(None, *wu0), (None, *wc0)], 50000, x_off=50)
    h_user1 = _gin_call(h_user, [(m_pu, *W(0, 'pu'))], N_USER)
    h_cat1 = _gin_call(h_cat, [(m_pc, *W(0, 'pc'))], N_CAT)

    (m_up1,) = seg_up(flat(h_user1), *e_up)
    (m_cp1,) = seg_cp(flat(h_cat1), *e_cp)

    wu1, wc1 = W(1, 'up'), W(1, 'cp')
    wo, bo = p['out_W'], p['out_b']
    out_a = _gin_out_call(h1p_a, [(m_up1, *wu1), (m_cp1, *wc1)], wo, bo, 5000)
    out_b = _gin_out_call(h1p_b, [(m_up1, *wu1), (None, *wc1)], wo, bo,
                          45000, m_off=5)
    out_c = _gin_out_call(h1p_c, [(None, *wu1), (None, *wc1)], wo, bo, 50000)
    return jnp.concatenate([out_a, out_b, out_c])
